# baseline (speedup 1.0000x reference)
import functools

import jax
import jax.numpy as jnp
from jax import lax
from jax.experimental import pallas as pl
from jax.experimental.pallas import tpu as pltpu
from jax.experimental.pallas import tpu_sc as plsc

_E = 8
_D = 1024
_DFF = 4096
_TMB = 2048
_CF = 512
_NCK = _DFF // _CF
_L = 16
_NV = _D // _L


def _sc_router_body(hs8_hbm, rw_hbm, xs_hbm, slots_hbm, nfir_hbm,
                    hs_v, rw_v, xs_v, sl_v, nf_v):
    cid = lax.axis_index("c")
    sid = lax.axis_index("s")

    @pl.when(jnp.logical_and(cid == 0, sid == 0))
    def _():
        pltpu.sync_copy(hs8_hbm, hs_v)
        pltpu.sync_copy(rw_hbm, rw_v)

        iota16 = lax.iota(jnp.int32, _L)

        def _perm(x, k):
            idx = jnp.bitwise_xor(iota16, k).reshape(_L, 1)
            return lax.gather(
                x, idx,
                lax.GatherDimensionNumbers(
                    offset_dims=(), collapsed_slice_dims=(0,),
                    start_index_map=(0,)),
                slice_sizes=(1,),
                mode=lax.GatherScatterMode.PROMISE_IN_BOUNDS)

        def _bf_sum(x):
            for k in (8, 4, 2, 1):
                x = x + _perm(x, k)
            return x

        def _bf_max(x):
            for k in (8, 4, 2, 1):
                x = jnp.maximum(x, _perm(x, k))
            return x

        def _bf_min(x):
            for k in (8, 4, 2, 1):
                x = jnp.minimum(x, _perm(x, k))
            return x

        scales = []
        for t in range(_E):
            def dot_step(i, accs):
                xc = hs_v[t, pl.ds(i * _L, _L)]
                return tuple(accs[e] + xc * rw_v[e, pl.ds(i * _L, _L)]
                             for e in range(_E))
            accs = lax.fori_loop(
                0, _NV, dot_step,
                tuple(jnp.zeros((_L,), jnp.float32) for _ in range(_E)))
            lv = jnp.full((_L,), -3e38, jnp.float32)
            for e in range(_E):
                lv = jnp.where(iota16 == e, _bf_sum(accs[e]), lv)
            m = _bf_max(lv)
            am = _bf_min(jnp.where(lv == m, iota16, jnp.full((_L,), _L,
                                                             jnp.int32)))
            sig = 1.0 / (1.0 + jnp.exp(-m))
            scale = jnp.where(am == 0, sig, jnp.zeros((_L,), jnp.float32))
            scales.append(scale)

            def scale_step(i, carry):
                xs_v[t, pl.ds(i * _L, _L)] = (
                    hs_v[t, pl.ds(i * _L, _L)] * scale)
                return carry
            lax.fori_loop(0, _NV, scale_step, jnp.int32(0))

        slots_vec = jnp.zeros((_L,), jnp.int32)
        n_vec = jnp.zeros((_L,), jnp.int32)
        last_vec = jnp.zeros((_L,), jnp.int32)
        one = jnp.full((_L,), 1, jnp.int32)
        zero = jnp.zeros((_L,), jnp.int32)
        for t in range(_E):
            fire_vec = scales[t] != 0.0
            hit = jnp.logical_and(fire_vec, iota16 == n_vec)
            slots_vec = slots_vec + jnp.where(
                hit, jnp.full((_L,), t, jnp.int32), zero)
            last_vec = jnp.where(fire_vec, jnp.full((_L,), t, jnp.int32),
                                 last_vec)
            n_vec = n_vec + jnp.where(fire_vec, one, zero)
        slots_vec = jnp.where(iota16 < n_vec, slots_vec, last_vec)
        sl_v[...] = slots_vec
        nf_v[...] = n_vec

        pltpu.sync_copy(xs_v, xs_hbm)
        pltpu.sync_copy(sl_v, slots_hbm)
        pltpu.sync_copy(nf_v, nfir_hbm)


@functools.partial(
    pl.kernel,
    mesh=plsc.VectorSubcoreMesh(core_axis_name="c", subcore_axis_name="s"),
    out_type=[
        jax.ShapeDtypeStruct((_E, _D), jnp.float32),
        jax.ShapeDtypeStruct((_L,), jnp.int32),
        jax.ShapeDtypeStruct((_L,), jnp.int32),
    ],
    scratch_types=[
        pltpu.VMEM((_E, _D), jnp.float32),
        pltpu.VMEM((_E, _D), jnp.float32),
        pltpu.VMEM((_E, _D), jnp.float32),
        pltpu.VMEM((_L,), jnp.int32),
        pltpu.VMEM((_L,), jnp.int32),
    ],
)
def _sc_router(hs8_hbm, rw_hbm, xs_hbm, slots_hbm, nfir_hbm,
               hs_v, rw_v, xs_v, sl_v, nf_v):
    _sc_router_body(hs8_hbm, rw_hbm, xs_hbm, slots_hbm, nfir_hbm,
                    hs_v, rw_v, xs_v, sl_v, nf_v)


def _experts_body(slots_ref, nfir_ref, xs_ref, wg_ref, wu_ref, wd_ref, v_ref):
    k = pl.program_id(0)
    c = pl.program_id(1)

    @pl.when(jnp.logical_and(k == 0, c == 0))
    def _():
        v_ref[...] = jnp.zeros_like(v_ref)

    @pl.when(k < nfir_ref[0])
    def _():
        x = xs_ref[0]
        g = jax.lax.dot_general(x, wg_ref[0], (((1,), (1,)), ((), ())),
                                preferred_element_type=jnp.float32)
        u = jax.lax.dot_general(x, wu_ref[0], (((1,), (1,)), ((), ())),
                                preferred_element_type=jnp.float32)
        a = g * jax.nn.sigmoid(g) * u
        pv = jax.lax.dot_general(a, wd_ref[0], (((1,), (1,)), ((), ())),
                                 preferred_element_type=jnp.float32)
        v_ref[0:1, :] += pv


def _we_chunk(k, c, slots, nfir):
    return jnp.where(k >= nfir[0], _NCK - 1, c)


def _wg_im(k, c, slots, nfir):
    return (slots[k], _we_chunk(k, c, slots, nfir), 0)


def _wd_im(k, c, slots, nfir):
    return (slots[k], 0, _we_chunk(k, c, slots, nfir))


def _xs_im(k, c, slots, nfir):
    return (slots[k], 0, 0)


def _shared_body(hs_ref, rw_ref, wg_ref, wu_ref, wd_ref, v_ref,
                 out_ref, scores_ref):
    fc = pl.program_id(1)

    @pl.when(fc == 0)
    def _():
        out_ref[...] = jnp.broadcast_to(v_ref[0:1, :], out_ref.shape)
        logits = jax.lax.dot_general(
            hs_ref[...], rw_ref[...], (((1,), (1,)), ((), ())),
            preferred_element_type=jnp.float32)
        amax = jnp.argmax(logits, axis=1)
        sig = jax.nn.sigmoid(jnp.max(logits, axis=1))
        eid = jax.lax.broadcasted_iota(jnp.int32, (_E, _TMB), 0)
        scores_ref[...] = jnp.where(eid == amax[None, :], sig[None, :], 0.0)

    x = hs_ref[...].astype(jnp.bfloat16)
    h = _CF // 2
    parts = []
    for j in range(2):
        wg = wg_ref[pl.ds(j * h, h), :].astype(jnp.bfloat16)
        wu = wu_ref[pl.ds(j * h, h), :].astype(jnp.bfloat16)
        wd = wd_ref[:, pl.ds(j * h, h)].astype(jnp.bfloat16)
        g = jax.lax.dot_general(x, wg, (((1,), (1,)), ((), ())),
                                preferred_element_type=jnp.float32)
        u = jax.lax.dot_general(x, wu, (((1,), (1,)), ((), ())),
                                preferred_element_type=jnp.float32)
        a = (g * jax.nn.sigmoid(g) * u).astype(jnp.bfloat16)
        parts.append(jax.lax.dot_general(
            a, wd, (((1,), (1,)), ((), ())),
            preferred_element_type=jnp.float32))
    out_ref[...] += parts[0] + parts[1]


def kernel(hidden_states, router_w, shared_wg, shared_wu, shared_wd,
           exp_wg, exp_wu, exp_wd):
    b, s, d = hidden_states.shape
    hs = hidden_states.reshape(-1, d)
    t = hs.shape[0]

    xs, slots, nfir = _sc_router(hs[:_E], router_w)
    xs = xs.reshape(_E, 1, _D)

    v8 = pl.pallas_call(
        _experts_body,
        grid_spec=pltpu.PrefetchScalarGridSpec(
            num_scalar_prefetch=2,
            grid=(_E, _NCK),
            in_specs=[
                pl.BlockSpec((1, 1, _D), _xs_im),
                pl.BlockSpec((1, _CF, _D), _wg_im),
                pl.BlockSpec((1, _CF, _D), _wg_im),
                pl.BlockSpec((1, _D, _CF), _wd_im),
            ],
            out_specs=pl.BlockSpec((_E, _D), lambda k, c, slots, nfir: (0, 0)),
        ),
        out_shape=jax.ShapeDtypeStruct((_E, _D), jnp.float32),
        compiler_params=pltpu.CompilerParams(
            dimension_semantics=("arbitrary", "arbitrary")),
    )(slots, nfir, xs, exp_wg, exp_wu, exp_wd)

    out, scores = pl.pallas_call(
        _shared_body,
        grid=(t // _TMB, _NCK),
        in_specs=[
            pl.BlockSpec((_TMB, _D), lambda i, f: (i, 0)),
            pl.BlockSpec((_E, _D), lambda i, f: (0, 0)),
            pl.BlockSpec((_CF, _D), lambda i, f: (f, 0)),
            pl.BlockSpec((_CF, _D), lambda i, f: (f, 0)),
            pl.BlockSpec((_D, _CF), lambda i, f: (0, f)),
            pl.BlockSpec((_E, _D), lambda i, f: (0, 0)),
        ],
        out_specs=[
            pl.BlockSpec((_TMB, _D), lambda i, f: (i, 0)),
            pl.BlockSpec((_E, _TMB), lambda i, f: (0, i)),
        ],
        out_shape=[
            jax.ShapeDtypeStruct((t, _D), jnp.float32),
            jax.ShapeDtypeStruct((_E, t), jnp.float32),
        ],
        compiler_params=pltpu.CompilerParams(
            dimension_semantics=("parallel", "arbitrary")),
    )(hs, router_w, shared_wg, shared_wu, shared_wd, v8)

    return out, scores
